# Initial kernel scaffold; baseline (speedup 1.0000x reference)
import jax
import jax.numpy as jnp
from jax.experimental import pallas as pl


def kernel(x, pos, knn_idx, W0, b0, g0, be0, W1, b1, g1, be1, W2, b2, g2, be2, W3, b3):
    raise NotImplementedError("write your pallas kernel here")


# speedup vs baseline: 3.7321x; 3.7321x over previous
import functools

import jax
import jax.numpy as jnp
from jax import lax
from jax.experimental import pallas as pl
from jax.experimental.pallas import tpu as pltpu
from jax.experimental.pallas import tpu_sc as plsc

_EPS = 1e-5
_PREC = lax.Precision.HIGHEST


_SC_NC = 2
_SC_NS = 16
_NW = _SC_NC * _SC_NS


def _sc_gather(table, idx):
    m = idx.shape[0]
    d = table.shape[1]
    per_w = m // _NW
    ch = 512
    n_ch = per_w // ch
    mesh = plsc.VectorSubcoreMesh(core_axis_name="c", subcore_axis_name="s")

    @functools.partial(
        pl.kernel,
        mesh=mesh,
        compiler_params=pltpu.CompilerParams(use_tc_tiling_on_sc=False),
        out_type=jax.ShapeDtypeStruct((m, d), jnp.float32),
        scratch_types=[
            pltpu.VMEM((per_w,), jnp.int32),
            pltpu.VMEM((ch, d), jnp.float32),
            pltpu.VMEM((ch, d), jnp.float32),
            pltpu.SemaphoreType.DMA,
            pltpu.SemaphoreType.DMA,
        ],
    )
    def k(table_hbm, idx_hbm, out_hbm, idx_v, buf0, buf1, sem0, sem1):
        wid = lax.axis_index("s") * _SC_NC + lax.axis_index("c")
        base = wid * per_w
        pltpu.sync_copy(idx_hbm.at[pl.ds(base, per_w)], idx_v)
        bufs = (buf0, buf1)
        sems = (sem0, sem1)
        cp = pltpu.async_copy(
            table_hbm.at[idx_v.at[pl.ds(0, ch)]], bufs[0], sems[0])
        for c in range(n_ch):
            nxt = None
            if c + 1 < n_ch:
                nxt = pltpu.async_copy(
                    table_hbm.at[idx_v.at[pl.ds((c + 1) * ch, ch)]],
                    bufs[(c + 1) % 2], sems[(c + 1) % 2])
            cp.wait()
            pltpu.sync_copy(bufs[c % 2], out_hbm.at[pl.ds(base + c * ch, ch)])
            cp = nxt

    return k(table, idx)


def _pre_body(x_ref, w_ref, b_ref, o_ref):
    o_ref[...] = (
        jnp.dot(x_ref[...], w_ref[...],
                preferred_element_type=jnp.float32, precision=_PREC)
        + b_ref[...])


def _stats0_body(g_ref, u_ref, o_ref, acc):
    j = pl.program_id(0)

    @pl.when(j == 0)
    def _():
        acc[...] = jnp.zeros_like(acc)

    h = g_ref[...] + u_ref[...]
    acc[0:1] += jnp.sum(h, axis=0, keepdims=True)
    acc[1:2] += jnp.sum(h * h, axis=0, keepdims=True)

    @pl.when(j == pl.num_programs(0) - 1)
    def _():
        o_ref[...] = acc[...]


def _affine(stats_ref, gb_ref, m_rows):
    s = stats_ref[...]
    mean = s[0:1] * (1.0 / m_rows)
    var = s[1:2] * (1.0 / m_rows) - mean * mean
    inv = lax.rsqrt(var + _EPS)
    a = inv * gb_ref[0:1]
    b = gb_ref[1:2] - mean * a
    return a, b


def _l1_body(m_rows, g_ref, u_ref, v1_ref, s0_ref, gb0_ref, w1_ref,
             o_ref, so_ref, acc):
    j = pl.program_id(0)
    a0, b0 = _affine(s0_ref, gb0_ref, m_rows)
    h0 = g_ref[...] + u_ref[...]
    h0n = jnp.maximum(h0 * a0 + b0, 0.0)
    h1 = (jnp.dot(h0n, w1_ref[...],
                  preferred_element_type=jnp.float32, precision=_PREC)
          + v1_ref[...])
    o_ref[...] = h1

    @pl.when(j == 0)
    def _():
        acc[...] = jnp.zeros_like(acc)

    acc[0:1] += jnp.sum(h1, axis=0, keepdims=True)
    acc[1:2] += jnp.sum(h1 * h1, axis=0, keepdims=True)

    @pl.when(j == pl.num_programs(0) - 1)
    def _():
        so_ref[...] = acc[...]


def _l2_body(m_rows, g_ref, u_ref, h1_ref, v2_ref, s0_ref, s1_ref,
             gb0_ref, gb1_ref, w2a_ref, w2b_ref, o_ref, so_ref, acc):
    j = pl.program_id(0)
    a0, b0 = _affine(s0_ref, gb0_ref, m_rows)
    a1, b1 = _affine(s1_ref, gb1_ref, m_rows)
    h0n = jnp.maximum((g_ref[...] + u_ref[...]) * a0 + b0, 0.0)
    h1n = jnp.maximum(h1_ref[...] * a1 + b1, 0.0)
    h2 = (jnp.dot(h1n, w2a_ref[...],
                  preferred_element_type=jnp.float32, precision=_PREC)
          + jnp.dot(h0n, w2b_ref[...],
                    preferred_element_type=jnp.float32, precision=_PREC)
          + v2_ref[...])
    o_ref[...] = h2

    @pl.when(j == 0)
    def _():
        acc[...] = jnp.zeros_like(acc)

    acc[0:1] += jnp.sum(h2, axis=0, keepdims=True)
    acc[1:2] += jnp.sum(h2 * h2, axis=0, keepdims=True)

    @pl.when(j == pl.num_programs(0) - 1)
    def _():
        so_ref[...] = acc[...]


def _l3_body(m_rows, g_ref, u_ref, h1_ref, h2_ref, v3_ref,
             s0_ref, s1_ref, s2_ref, gb0_ref, gb1_ref, gb2_ref,
             w3a_ref, w3b_ref, w3c_ref, o3_ref, o2_ref, o1_ref, o0_ref):
    k = pl.program_id(1)
    a0, b0 = _affine(s0_ref, gb0_ref, m_rows)
    a1, b1 = _affine(s1_ref, gb1_ref, m_rows)
    a2, b2 = _affine(s2_ref, gb2_ref, m_rows)
    h0n = jnp.maximum((g_ref[...] + u_ref[...]) * a0 + b0, 0.0)
    h1n = jnp.maximum(h1_ref[...] * a1 + b1, 0.0)
    h2n = jnp.maximum(h2_ref[...] * a2 + b2, 0.0)
    h3 = (jnp.dot(h2n, w3a_ref[...],
                  preferred_element_type=jnp.float32, precision=_PREC)
          + jnp.dot(h1n, w3b_ref[...],
                    preferred_element_type=jnp.float32, precision=_PREC)
          + jnp.dot(h0n, w3c_ref[...],
                    preferred_element_type=jnp.float32, precision=_PREC)
          + v3_ref[...])

    @pl.when(k == 0)
    def _():
        o3_ref[...] = h3
        o2_ref[...] = h2n
        o1_ref[...] = h1n
        o0_ref[...] = h0n

    @pl.when(k > 0)
    def _():
        o3_ref[...] = jnp.maximum(o3_ref[...], h3)
        o2_ref[...] = jnp.maximum(o2_ref[...], h2n)
        o1_ref[...] = jnp.maximum(o1_ref[...], h1n)
        o0_ref[...] = jnp.maximum(o0_ref[...], h0n)


def kernel(x, pos, knn_idx, W0, b0, g0, be0, W1, b1, g1, be1, W2, b2, g2,
           be2, W3, b3):
    del pos
    B, N, C = x.shape
    K = knn_idx.shape[2]
    G = W0.shape[1]
    P = B * N
    M = P * K
    f32 = jnp.float32

    x2d = x.reshape(P, C)

    w0x, w0n, w0d = W0[:C], W0[C:2 * C], W0[2 * C:]
    wz = w0n + w0d
    wu = w0x - w0d
    w1a, w1x = W1[:G], W1[G:]
    w2a, w2b, w2x = W2[:G], W2[G:2 * G], W2[2 * G:]
    w3a, w3b, w3c, w3x = W3[:G], W3[G:2 * G], W3[2 * G:3 * G], W3[3 * G:]
    wcat = jnp.concatenate([wz, wu, w1x, w2x, w3x], axis=1)
    bcat = jnp.concatenate(
        [jnp.zeros_like(b0), b0, b1, b2, b3]).reshape(1, 5 * G)
    gb0 = jnp.stack([g0, be0])
    gb1 = jnp.stack([g1, be1])
    gb2 = jnp.stack([g2, be2])

    RP = 4096
    pre = pl.pallas_call(
        _pre_body,
        grid=(P // RP,),
        in_specs=[
            pl.BlockSpec((RP, C), lambda j: (j, 0)),
            pl.BlockSpec((C, 5 * G), lambda j: (0, 0)),
            pl.BlockSpec((1, 5 * G), lambda j: (0, 0)),
        ],
        out_specs=pl.BlockSpec((RP, 5 * G), lambda j: (j, 0)),
        out_shape=jax.ShapeDtypeStruct((P, 5 * G), f32),
    )(x2d, wcat, bcat)
    z0 = pre[:, :G]
    u0 = pre[:, G:2 * G]
    v1 = pre[:, 2 * G:3 * G]
    v2 = pre[:, 3 * G:4 * G]
    v3 = pre[:, 4 * G:]

    idxf = (knn_idx + (jnp.arange(B, dtype=jnp.int32) * N)[:, None, None])
    idxf = idxf.transpose(2, 0, 1).reshape(M)

    g0rows = _sc_gather(z0, idxf)

    R = 2048
    nb = M // R
    npb = P // R

    def espec(width=G):
        return pl.BlockSpec((R, width), lambda j: (j, 0))

    def pspec():
        return pl.BlockSpec((R, G), lambda j: (lax.rem(j, npb), 0))

    def wspec(shape):
        return pl.BlockSpec(shape, lambda j: (0, 0))

    stats0 = pl.pallas_call(
        _stats0_body,
        grid=(nb,),
        in_specs=[espec(), pspec()],
        out_specs=pl.BlockSpec((2, G), lambda j: (0, 0)),
        out_shape=jax.ShapeDtypeStruct((2, G), f32),
        scratch_shapes=[pltpu.VMEM((2, G), f32)],
    )(g0rows, u0)

    h1, stats1 = pl.pallas_call(
        functools.partial(_l1_body, float(M)),
        grid=(nb,),
        in_specs=[espec(), pspec(), pspec(), wspec((2, G)), wspec((2, G)),
                  wspec((G, G))],
        out_specs=[espec(), pl.BlockSpec((2, G), lambda j: (0, 0))],
        out_shape=[jax.ShapeDtypeStruct((M, G), f32),
                   jax.ShapeDtypeStruct((2, G), f32)],
        scratch_shapes=[pltpu.VMEM((2, G), f32)],
    )(g0rows, u0, v1, stats0, gb0, w1a)

    h2, stats2 = pl.pallas_call(
        functools.partial(_l2_body, float(M)),
        grid=(nb,),
        in_specs=[espec(), pspec(), espec(), pspec(), wspec((2, G)),
                  wspec((2, G)), wspec((2, G)), wspec((2, G)),
                  wspec((G, G)), wspec((G, G))],
        out_specs=[espec(), pl.BlockSpec((2, G), lambda j: (0, 0))],
        out_shape=[jax.ShapeDtypeStruct((M, G), f32),
                   jax.ShapeDtypeStruct((2, G), f32)],
        scratch_shapes=[pltpu.VMEM((2, G), f32)],
    )(g0rows, u0, h1, v2, stats0, stats1, gb0, gb1, w2a, w2b)

    RQ = 2048
    nq = P // RQ

    def eqspec():
        return pl.BlockSpec((RQ, G), lambda pb, k: (k * nq + pb, 0))

    def pqspec():
        return pl.BlockSpec((RQ, G), lambda pb, k: (pb, 0))

    def wqspec(shape):
        return pl.BlockSpec(shape, lambda pb, k: (0, 0))

    outs = pl.pallas_call(
        functools.partial(_l3_body, float(M)),
        grid=(nq, K),
        in_specs=[eqspec(), pqspec(), eqspec(), eqspec(), pqspec(),
                  wqspec((2, G)), wqspec((2, G)), wqspec((2, G)),
                  wqspec((2, G)), wqspec((2, G)), wqspec((2, G)),
                  wqspec((G, G)), wqspec((G, G)), wqspec((G, G))],
        out_specs=[pqspec(), pqspec(), pqspec(), pqspec()],
        out_shape=[jax.ShapeDtypeStruct((P, G), f32)] * 4,
    )(g0rows, u0, h1, h2, v3, stats0, stats1, stats2, gb0, gb1, gb2,
      w3a, w3b, w3c)
    o3, o2, o1, o0 = outs

    y = jnp.concatenate([o3, o2, o1, o0, x2d], axis=1).reshape(B, N, 5 * G)
    return (y, knn_idx)
